# baseline (speedup 1.0000x reference)
import jax
import jax.numpy as jnp
from jax import lax
from jax.experimental import pallas as pl
from jax.experimental.pallas import tpu as pltpu
from jax.experimental.pallas import tpu_sc as plsc

_NUM_REG = 4
_NUM_CAT = 26
_VOCAB = 100000
_DIM = 32
_BATCH = 16384
_NF = _NUM_REG + _NUM_CAT

_NC = 2
_NS = 16
_NW = _NC * _NS
_L = 16

_NWH = _NS
_B_PER_W = (_BATCH // 2) // _NWH
_NB = 32
_NCHUNK = _B_PER_W // _NB
_ROWS = _NB * _NF
_NVEC = _ROWS // _L
_NSUB = 10
_SUB = _ROWS // _NSUB


def _body(inp_hbm, tables_hbm, wreg_hbm, breg_hbm, out_hbm,
          inp_v, idx_v, rows_v, ofs_v, wb_v, gsem, osem):
    wid = lax.axis_index("s")
    w0 = wid * _B_PER_W * _NF

    pltpu.sync_copy(wreg_hbm, wb_v.at[0])
    pltpu.sync_copy(breg_hbm, wb_v.at[1])

    lane = lax.broadcasted_iota(jnp.int32, (_L,), 0)

    def ofsvec(k, ivec):
        ofs = jnp.where(ivec < _NUM_CAT, ivec * _VOCAB, -1)
        ofs_v[pl.ds(k * _L, _L)] = ofs
        nxt = ivec + _L
        return jnp.where(nxt >= _NF, nxt - _NF, nxt)

    lax.fori_loop(0, _NVEC, ofsvec, lane)

    wlo = [wb_v[0, j, pl.ds(0, _L)] for j in range(_NUM_REG)]
    whi = [wb_v[0, j, pl.ds(_L, _L)] for j in range(_NUM_REG)]
    blo = [wb_v[1, j, pl.ds(0, _L)] for j in range(_NUM_REG)]
    bhi = [wb_v[1, j, pl.ds(_L, _L)] for j in range(_NUM_REG)]

    def stage(c, buf):
        p0 = w0 + c * _ROWS
        pltpu.sync_copy(inp_hbm.at[pl.ds(p0, _ROWS)], inp_v.at[buf])

        def ivec(k, carry):
            base = k * _L
            ofs = ofs_v[pl.ds(base, _L)]
            vals = inp_v[buf, pl.ds(base + _NUM_REG, _L)]
            idx = jnp.where(ofs < 0, 0, ofs + vals.astype(jnp.int32))
            idx_v[buf, pl.ds(base, _L)] = idx
            return carry

        lax.fori_loop(0, _NVEC, ivec, 0)

        def grow(k, carry):
            idx16 = idx_v[buf, pl.ds(k * _L, _L)]
            base = k * _L
            for l in range(_L):
                pltpu.async_copy(
                    tables_hbm.at[pl.ds(idx16[l], 1)],
                    rows_v.at[buf, pl.ds(base + l, 1)],
                    gsem.at[buf],
                )
            return carry

        lax.fori_loop(0, _NVEC, grow, 0)

    def finish(c, buf, gcopies):
        del gcopies
        pltpu.make_async_copy(
            tables_hbm.at[pl.ds(0, _ROWS)], rows_v.at[buf], gsem.at[buf]
        ).wait()

        def regrow(b, carry):
            xs = inp_v[buf, pl.ds(b * _NF, _L)]
            r = b * _NF + _NUM_CAT
            for j in range(_NUM_REG):
                x = xs[j]
                rows_v[buf, r + j, pl.ds(0, _L)] = x * wlo[j] + blo[j]
                rows_v[buf, r + j, pl.ds(_L, _L)] = x * whi[j] + bhi[j]
            return carry

        lax.fori_loop(0, _NB, regrow, 0)
        p0 = w0 + c * _ROWS
        return pltpu.async_copy(rows_v.at[buf], out_hbm.at[pl.ds(p0, _ROWS)], osem)

    gcopies = [None, None]
    ocopies = [None, None]
    gcopies[0] = stage(0, 0)
    for c in range(_NCHUNK):
        buf = c % 2
        nbuf = (c + 1) % 2
        if c + 1 < _NCHUNK:
            if ocopies[nbuf] is not None:
                ocopies[nbuf].wait()
            gcopies[nbuf] = stage(c + 1, nbuf)
        ocopies[buf] = finish(c, buf, gcopies[buf])
    for oc in ocopies:
        if oc is not None:
            oc.wait()


@jax.jit
def kernel(all_inputs, tables, Wreg, breg):
    inp_flat = all_inputs.reshape(_BATCH * _NF)
    tables_flat = tables.reshape(_NUM_CAT * _VOCAB, _DIM)

    mesh = plsc.VectorSubcoreMesh(
        core_axis_name="c", subcore_axis_name="s", num_cores=1
    )
    half_rows = (_BATCH // 2) * _NF
    call = pl.kernel(
        _body,
        out_type=jax.ShapeDtypeStruct((half_rows, _DIM), jnp.float32),
        mesh=mesh,
        scratch_types=[
            pltpu.VMEM((2, _ROWS), jnp.float32),
            pltpu.VMEM((2, _ROWS), jnp.int32),
            pltpu.VMEM((2, _ROWS, _DIM), jnp.float32),
            pltpu.VMEM((_ROWS,), jnp.int32),
            pltpu.VMEM((2, _NUM_REG, _DIM), jnp.float32),
            pltpu.SemaphoreType.DMA((2,)),
            pltpu.SemaphoreType.DMA,
        ],
        compiler_params=pltpu.CompilerParams(use_tc_tiling_on_sc=False),
    )
    out0 = call(inp_flat[:half_rows], tables_flat, Wreg, breg)
    out1 = call(inp_flat[half_rows:], tables_flat, Wreg, breg)
    out = jnp.concatenate([out0, out1], axis=0)
    return out.reshape(_BATCH, _NF, _DIM)


# speedup vs baseline: 1.8105x; 1.8105x over previous
import jax
import jax.numpy as jnp
from jax import lax
from jax.experimental import pallas as pl
from jax.experimental.pallas import tpu as pltpu
from jax.experimental.pallas import tpu_sc as plsc

_NUM_REG = 4
_NUM_CAT = 26
_VOCAB = 100000
_DIM = 32
_BATCH = 16384
_NF = _NUM_REG + _NUM_CAT

_NC = 2
_NS = 16
_NW = _NC * _NS
_L = 16

_B_PER_W = _BATCH // _NW
_NB = 32
_NCHUNK = _B_PER_W // _NB
_ROWS = _NB * _NF


def _body(inp_hbm, tables_hbm, wreg_hbm, breg_hbm, out_hbm,
          inp_v, rows_v, regs_v, wb_v, gsem, osem):
    wid = lax.axis_index("s") * _NC + lax.axis_index("c")
    b_w = wid * _B_PER_W

    pltpu.sync_copy(wreg_hbm, wb_v.at[0])
    pltpu.sync_copy(breg_hbm, wb_v.at[1])

    lane = lax.broadcasted_iota(jnp.int32, (_L,), 0)
    ofs0 = lane * _VOCAB
    ofs1 = (lane + 10) * _VOCAB

    wlo = [wb_v[0, j, pl.ds(0, _L)] for j in range(_NUM_REG)]
    whi = [wb_v[0, j, pl.ds(_L, _L)] for j in range(_NUM_REG)]
    blo = [wb_v[1, j, pl.ds(0, _L)] for j in range(_NUM_REG)]
    bhi = [wb_v[1, j, pl.ds(_L, _L)] for j in range(_NUM_REG)]

    def stage(c, buf):
        p0 = (b_w + c * _NB) * _NF
        pltpu.sync_copy(inp_hbm.at[pl.ds(p0, _ROWS)], inp_v.at[buf])

        def grow(b, carry):
            v0 = inp_v[buf, pl.ds(b * _NF + 4, _L)].astype(jnp.int32) + ofs0
            v1 = inp_v[buf, pl.ds(b * _NF + 14, _L)].astype(jnp.int32) + ofs1
            for i in range(_NUM_CAT):
                idx = v0[i] if i < _L else v1[i - 10]
                pltpu.async_copy(
                    tables_hbm.at[pl.ds(idx, 1)],
                    rows_v.at[buf, b, pl.ds(i, 1)],
                    gsem.at[buf],
                )
            return carry

        lax.fori_loop(0, _NB, grow, 0)

    def finish(c, buf):
        pltpu.make_async_copy(
            out_hbm.at[pl.ds(0, _NB), pl.ds(0, _NUM_CAT)],
            rows_v.at[buf],
            gsem.at[buf],
        ).wait()

        def regrow(b, carry):
            xs = inp_v[buf, pl.ds(b * _NF, _L)]
            for j in range(_NUM_REG):
                x = xs[j]
                regs_v[buf, b, j, pl.ds(0, _L)] = x * wlo[j] + blo[j]
                regs_v[buf, b, j, pl.ds(_L, _L)] = x * whi[j] + bhi[j]
            return carry

        lax.fori_loop(0, _NB, regrow, 0)
        b0 = b_w + c * _NB
        oc = pltpu.async_copy(
            rows_v.at[buf], out_hbm.at[pl.ds(b0, _NB), pl.ds(0, _NUM_CAT)], osem
        )
        or_ = pltpu.async_copy(
            regs_v.at[buf], out_hbm.at[pl.ds(b0, _NB), pl.ds(_NUM_CAT, _NUM_REG)],
            osem,
        )
        return (oc, or_)

    ocopies = [None, None]
    stage(0, 0)
    for c in range(_NCHUNK):
        buf = c % 2
        nbuf = (c + 1) % 2
        if c + 1 < _NCHUNK:
            if ocopies[nbuf] is not None:
                for oc in ocopies[nbuf]:
                    oc.wait()
            stage(c + 1, nbuf)
        ocopies[buf] = finish(c, buf)
    for ocs in ocopies:
        if ocs is not None:
            for oc in ocs:
                oc.wait()


@jax.jit
def kernel(all_inputs, tables, Wreg, breg):
    inp_flat = all_inputs.reshape(_BATCH * _NF)
    tables_flat = tables.reshape(_NUM_CAT * _VOCAB, _DIM)

    mesh = plsc.VectorSubcoreMesh(core_axis_name="c", subcore_axis_name="s")
    return pl.kernel(
        _body,
        out_type=jax.ShapeDtypeStruct((_BATCH, _NF, _DIM), jnp.float32),
        mesh=mesh,
        scratch_types=[
            pltpu.VMEM((2, _ROWS), jnp.float32),
            pltpu.VMEM((2, _NB, _NUM_CAT, _DIM), jnp.float32),
            pltpu.VMEM((2, _NB, _NUM_REG, _DIM), jnp.float32),
            pltpu.VMEM((2, _NUM_REG, _DIM), jnp.float32),
            pltpu.SemaphoreType.DMA((2,)),
            pltpu.SemaphoreType.DMA,
        ],
        compiler_params=pltpu.CompilerParams(use_tc_tiling_on_sc=False),
    )(inp_flat, tables_flat, Wreg, breg)


# speedup vs baseline: 1.8211x; 1.0059x over previous
import jax
import jax.numpy as jnp
from jax import lax
from jax.experimental import pallas as pl
from jax.experimental.pallas import tpu as pltpu
from jax.experimental.pallas import tpu_sc as plsc

_NUM_REG = 4
_NUM_CAT = 26
_VOCAB = 100000
_DIM = 32
_BATCH = 16384
_NF = _NUM_REG + _NUM_CAT

_NC = 2
_NS = 16
_NW = _NC * _NS
_L = 16

_B_PER_W = _BATCH // _NW
_NB = 64
_NCHUNK = _B_PER_W // _NB
_ROWS = _NB * _NF


def _body(inp_hbm, tables_hbm, wreg_hbm, breg_hbm, out_hbm,
          inp_v, rows_v, regs_v, wb_v, gsem, osem):
    wid = lax.axis_index("s") * _NC + lax.axis_index("c")
    b_w = wid * _B_PER_W

    pltpu.sync_copy(wreg_hbm, wb_v.at[0])
    pltpu.sync_copy(breg_hbm, wb_v.at[1])

    lane = lax.broadcasted_iota(jnp.int32, (_L,), 0)
    ofs0 = lane * _VOCAB
    ofs1 = (lane + 10) * _VOCAB

    wlo = [wb_v[0, j, pl.ds(0, _L)] for j in range(_NUM_REG)]
    whi = [wb_v[0, j, pl.ds(_L, _L)] for j in range(_NUM_REG)]
    blo = [wb_v[1, j, pl.ds(0, _L)] for j in range(_NUM_REG)]
    bhi = [wb_v[1, j, pl.ds(_L, _L)] for j in range(_NUM_REG)]

    def stage(c, buf):
        p0 = (b_w + c * _NB) * _NF
        pltpu.sync_copy(inp_hbm.at[pl.ds(p0, _ROWS)], inp_v.at[buf])

        def grow(b, carry):
            v0 = inp_v[buf, pl.ds(b * _NF + 4, _L)].astype(jnp.int32) + ofs0
            v1 = inp_v[buf, pl.ds(b * _NF + 14, _L)].astype(jnp.int32) + ofs1
            for i in range(_NUM_CAT):
                idx = v0[i] if i < _L else v1[i - 10]
                pltpu.async_copy(
                    tables_hbm.at[pl.ds(idx, 1)],
                    rows_v.at[buf, b, pl.ds(i, 1)],
                    gsem.at[buf],
                )
            return carry

        lax.fori_loop(0, _NB, grow, 0)

    def finish(c, buf):
        pltpu.make_async_copy(
            out_hbm.at[pl.ds(0, _NB), pl.ds(0, _NUM_CAT)],
            rows_v.at[buf],
            gsem.at[buf],
        ).wait()

        def regrow(b, carry):
            xs = inp_v[buf, pl.ds(b * _NF, _L)]
            for j in range(_NUM_REG):
                x = xs[j]
                regs_v[buf, b, j, pl.ds(0, _L)] = x * wlo[j] + blo[j]
                regs_v[buf, b, j, pl.ds(_L, _L)] = x * whi[j] + bhi[j]
            return carry

        lax.fori_loop(0, _NB, regrow, 0)
        b0 = b_w + c * _NB
        oc = pltpu.async_copy(
            rows_v.at[buf], out_hbm.at[pl.ds(b0, _NB), pl.ds(0, _NUM_CAT)], osem
        )
        or_ = pltpu.async_copy(
            regs_v.at[buf], out_hbm.at[pl.ds(b0, _NB), pl.ds(_NUM_CAT, _NUM_REG)],
            osem,
        )
        return (oc, or_)

    ocopies = [None, None]
    stage(0, 0)
    for c in range(_NCHUNK):
        buf = c % 2
        nbuf = (c + 1) % 2
        if c + 1 < _NCHUNK:
            if ocopies[nbuf] is not None:
                for oc in ocopies[nbuf]:
                    oc.wait()
            stage(c + 1, nbuf)
        ocopies[buf] = finish(c, buf)
    for ocs in ocopies:
        if ocs is not None:
            for oc in ocs:
                oc.wait()


@jax.jit
def kernel(all_inputs, tables, Wreg, breg):
    inp_flat = all_inputs.reshape(_BATCH * _NF)
    tables_flat = tables.reshape(_NUM_CAT * _VOCAB, _DIM)

    mesh = plsc.VectorSubcoreMesh(core_axis_name="c", subcore_axis_name="s")
    return pl.kernel(
        _body,
        out_type=jax.ShapeDtypeStruct((_BATCH, _NF, _DIM), jnp.float32),
        mesh=mesh,
        scratch_types=[
            pltpu.VMEM((2, _ROWS), jnp.float32),
            pltpu.VMEM((2, _NB, _NUM_CAT, _DIM), jnp.float32),
            pltpu.VMEM((2, _NB, _NUM_REG, _DIM), jnp.float32),
            pltpu.VMEM((2, _NUM_REG, _DIM), jnp.float32),
            pltpu.SemaphoreType.DMA((2,)),
            pltpu.SemaphoreType.DMA,
        ],
        compiler_params=pltpu.CompilerParams(use_tc_tiling_on_sc=False),
    )(inp_flat, tables_flat, Wreg, breg)
